# Initial kernel scaffold; baseline (speedup 1.0000x reference)
import jax
import jax.numpy as jnp
from jax.experimental import pallas as pl


def kernel(features, nodes, neigh_idx, weight):
    raise NotImplementedError("write your pallas kernel here")


# speedup vs baseline: 2.0585x; 2.0585x over previous
import functools

import jax
import jax.numpy as jnp
from jax import lax
from jax.experimental import pallas as pl
from jax.experimental.pallas import tpu as pltpu
from jax.experimental.pallas import tpu_sc as plsc

_B = 16384
_D = 256
_E = 256
_S = 11
_ROWS = _B * _S
_NC = 2
_NS = 16
_NW = _NC * _NS
_RPW = _ROWS // _NW
_CH = 128
_NCH = _RPW // _CH

_BT = 512
_NBT = _B // _BT

_sc_mesh = plsc.VectorSubcoreMesh(core_axis_name="c", subcore_axis_name="s")


@functools.partial(
    pl.kernel,
    mesh=_sc_mesh,
    out_type=jax.ShapeDtypeStruct((_ROWS, _D), jnp.float32),
    scratch_types=[
        pltpu.VMEM((_CH,), jnp.int32),
        pltpu.VMEM((_CH, _D), jnp.float32),
        pltpu.SemaphoreType.DMA,
    ],
)
def _sc_gather(table_hbm, idx_hbm, out_hbm, idx_v, rows_v, sem):
    wid = lax.axis_index("s") * _NC + lax.axis_index("c")
    base = wid * _RPW

    @pl.loop(0, _NCH)
    def _(g):
        off = base + g * _CH
        pltpu.sync_copy(idx_hbm.at[pl.ds(off, _CH)], idx_v)
        pltpu.async_copy(table_hbm.at[idx_v], rows_v, sem).wait()
        pltpu.sync_copy(rows_v, out_hbm.at[pl.ds(off, _CH)])


def _tc_body(g_ref, w_ref, o_ref):
    g = g_ref[...]
    self_f = g[:, 0, :]
    total = jnp.sum(g, axis=1)
    neigh = (total - self_f) * jnp.float32(0.1)
    w = w_ref[...]
    acc = lax.dot_general(
        w[:, :_D], self_f, (((1,), (1,)), ((), ())),
        preferred_element_type=jnp.float32, precision=lax.Precision.HIGHEST)
    acc = acc + lax.dot_general(
        w[:, _D:], neigh, (((1,), (1,)), ((), ())),
        preferred_element_type=jnp.float32, precision=lax.Precision.HIGHEST)
    o_ref[...] = jnp.maximum(acc, jnp.float32(0.0))


def _tc_matmul(gathered, weight):
    return pl.pallas_call(
        _tc_body,
        grid=(_NBT,),
        in_specs=[
            pl.BlockSpec((_BT, _S, _D), lambda i: (i, 0, 0)),
            pl.BlockSpec((_E, 2 * _D), lambda i: (0, 0)),
        ],
        out_specs=pl.BlockSpec((_E, _BT), lambda i: (0, i)),
        out_shape=jax.ShapeDtypeStruct((_E, _B), jnp.float32),
    )(gathered, weight)


def kernel(features, nodes, neigh_idx, weight):
    idx_all = jnp.concatenate([nodes[:, None], neigh_idx], axis=1).reshape(-1)
    gathered = _sc_gather(features, idx_all)
    gathered = gathered.reshape(_B, _S, _D)
    return _tc_matmul(gathered, weight)


# speedup vs baseline: 2.2571x; 1.0965x over previous
import functools

import jax
import jax.numpy as jnp
from jax import lax
from jax.experimental import pallas as pl
from jax.experimental.pallas import tpu as pltpu
from jax.experimental.pallas import tpu_sc as plsc

_B = 16384
_D = 256
_E = 256
_S = 11
_ROWS = _B * _S
_NC = 2
_NS = 16
_NW = _NC * _NS
_RPW = _ROWS // _NW
_CH = 128
_NCH = _RPW // _CH

_BT = 512
_NBT = _B // _BT

_sc_mesh = plsc.VectorSubcoreMesh(core_axis_name="c", subcore_axis_name="s")


@functools.partial(
    pl.kernel,
    mesh=_sc_mesh,
    out_type=jax.ShapeDtypeStruct((_ROWS, _D), jnp.float32),
    scratch_types=[
        pltpu.VMEM((_RPW,), jnp.int32),
        pltpu.VMEM((2, _CH, _D), jnp.float32),
        pltpu.SemaphoreType.DMA,
        pltpu.SemaphoreType.DMA,
        pltpu.SemaphoreType.DMA,
        pltpu.SemaphoreType.DMA,
    ],
)
def _sc_gather(table_hbm, idx_hbm, out_hbm, idx_v, bufs, g0, g1, w0, w1):
    wid = lax.axis_index("s") * _NC + lax.axis_index("c")
    base = wid * _RPW
    pltpu.sync_copy(idx_hbm.at[pl.ds(base, _RPW)], idx_v)

    def gather(c, p, sem):
        return pltpu.make_async_copy(
            table_hbm.at[idx_v.at[pl.ds(c * _CH, _CH)]], bufs.at[p], sem)

    def writeback(c, p, sem):
        return pltpu.make_async_copy(
            bufs.at[p], out_hbm.at[pl.ds(base + c * _CH, _CH)], sem)

    gather(0, 0, g0).start()
    gather(1, 1, g1).start()

    @pl.loop(0, _NCH // 2)
    def _(i):
        for p, gs, ws in ((0, g0, w0), (1, g1, w1)):
            c = i * 2 + p
            gather(c, p, gs).wait()
            writeback(c, p, ws).start()
            writeback(c, p, ws).wait()
            nc = c + 2

            @pl.when(nc < _NCH)
            def _():
                gather(nc, p, gs).start()


def _tc_body(g_ref, w_ref, o_ref):
    g = g_ref[...]
    self_f = g[:, 0, :]
    total = jnp.sum(g, axis=1)
    neigh = (total - self_f) * jnp.float32(0.1)
    w = w_ref[...]
    acc = lax.dot_general(
        w[:, :_D], self_f, (((1,), (1,)), ((), ())),
        preferred_element_type=jnp.float32, precision=lax.Precision.HIGHEST)
    acc = acc + lax.dot_general(
        w[:, _D:], neigh, (((1,), (1,)), ((), ())),
        preferred_element_type=jnp.float32, precision=lax.Precision.HIGHEST)
    o_ref[...] = jnp.maximum(acc, jnp.float32(0.0))


def _tc_matmul(gathered, weight):
    return pl.pallas_call(
        _tc_body,
        grid=(_NBT,),
        in_specs=[
            pl.BlockSpec((_BT, _S, _D), lambda i: (i, 0, 0)),
            pl.BlockSpec((_E, 2 * _D), lambda i: (0, 0)),
        ],
        out_specs=pl.BlockSpec((_E, _BT), lambda i: (0, i)),
        out_shape=jax.ShapeDtypeStruct((_E, _B), jnp.float32),
    )(gathered, weight)


def kernel(features, nodes, neigh_idx, weight):
    idx_all = jnp.concatenate([nodes[:, None], neigh_idx], axis=1).reshape(-1)
    gathered = _sc_gather(features, idx_all)
    gathered = gathered.reshape(_B, _S, _D)
    return _tc_matmul(gathered, weight)


# speedup vs baseline: 5.3736x; 2.3808x over previous
import functools

import jax
import jax.numpy as jnp
from jax import lax
from jax.experimental import pallas as pl
from jax.experimental.pallas import tpu as pltpu
from jax.experimental.pallas import tpu_sc as plsc

_B = 16384
_D = 256
_E = 256
_NC = 2
_NS = 16
_NW = _NC * _NS
_BPT = _B // _NW
_C = 8
_NSTEP = _BPT // _C
_SCH = 128
_NSCH = _BPT // _SCH

_BT = 2048
_NBT = _B // _BT

_sc_mesh = plsc.VectorSubcoreMesh(core_axis_name="c", subcore_axis_name="s")


@functools.partial(
    pl.kernel,
    mesh=_sc_mesh,
    out_type=(
        jax.ShapeDtypeStruct((_B, _D), jnp.float32),
        jax.ShapeDtypeStruct((_B, _D), jnp.float32),
    ),
    scratch_types=[
        pltpu.VMEM((_BPT * 10,), jnp.int32),
        pltpu.VMEM((_BPT,), jnp.int32),
        pltpu.VMEM((2, _SCH, _D), jnp.float32),
        pltpu.VMEM((2, _C, _D), jnp.float32),
        pltpu.SemaphoreType.DMA,
        pltpu.SemaphoreType.DMA,
        pltpu.SemaphoreType.DMA,
        pltpu.SemaphoreType.DMA,
    ],
)
def _sc_gather_sum(table_hbm, nidx_hbm, sidx_hbm,
                   self_hbm, nsum_hbm,
                   nidx_v, sidx_v, bufs, obuf,
                   g0, g1, o0, o1):
    cid = lax.axis_index("c")
    sid = lax.axis_index("s")
    wid = sid * _NC + cid

    pltpu.sync_copy(nidx_hbm.at[pl.ds(wid * _BPT * 10, _BPT * 10)], nidx_v)
    pltpu.sync_copy(sidx_hbm.at[pl.ds(wid * _BPT, _BPT)], sidx_v)

    def ngather(c, p, sem):
        return pltpu.make_async_copy(
            table_hbm.at[nidx_v.at[pl.ds(c * (10 * _C), 10 * _C)]],
            bufs.at[p, pl.ds(0, 10 * _C)], sem)

    def copyout(c, p, sem):
        return pltpu.make_async_copy(
            obuf.at[p],
            nsum_hbm.at[pl.ds(wid * _BPT + c * _C, _C)], sem)

    def sum_rows(p):
        buf = bufs.at[p]
        ob = obuf.at[p]

        @pl.loop(0, _C)
        def _(b):
            r0 = b * 10
            for k in range(_D // 16):
                sl = pl.ds(k * 16, 16)
                v = buf[r0, sl]
                for s in range(1, 10):
                    v = v + buf[r0 + s, sl]
                ob[b, sl] = v

    ngather(0, 0, g0).start()
    ngather(1, 1, g1).start()

    @pl.loop(0, _NSTEP // 2)
    def _(i):
        for p, gs, os in ((0, g0, o0), (1, g1, o1)):
            c = i * 2 + p
            ngather(c, p, gs).wait()

            @pl.when(i > 0)
            def _():
                copyout(c - 2, p, os).wait()

            sum_rows(p)
            copyout(c, p, os).start()
            nc = c + 2

            @pl.when(nc < _NSTEP)
            def _():
                ngather(nc, p, gs).start()

    copyout(_NSTEP - 2, 0, o0).wait()
    copyout(_NSTEP - 1, 1, o1).wait()

    def sgather(c, p, sem):
        return pltpu.make_async_copy(
            table_hbm.at[sidx_v.at[pl.ds(c * _SCH, _SCH)]], bufs.at[p], sem)

    def swrite(c, p, sem):
        return pltpu.make_async_copy(
            bufs.at[p], self_hbm.at[pl.ds(wid * _BPT + c * _SCH, _SCH)], sem)

    sgather(0, 0, g0).start()
    sgather(1, 1, g1).start()

    @pl.loop(0, _NSCH // 2)
    def _(i):
        for p, gs, ws in ((0, g0, o0), (1, g1, o1)):
            c = i * 2 + p
            sgather(c, p, gs).wait()
            swrite(c, p, ws).start()
            swrite(c, p, ws).wait()
            nc = c + 2

            @pl.when(nc < _NSCH)
            def _():
                sgather(nc, p, gs).start()


def _tc_body(s_ref, n_ref, w_ref, o_ref):
    self_f = s_ref[...]
    neigh = n_ref[...] * jnp.float32(0.1)
    w = w_ref[...]
    acc = lax.dot_general(
        w[:, :_D], self_f, (((1,), (1,)), ((), ())),
        preferred_element_type=jnp.float32, precision=lax.Precision.HIGHEST)
    acc = acc + lax.dot_general(
        w[:, _D:], neigh, (((1,), (1,)), ((), ())),
        preferred_element_type=jnp.float32, precision=lax.Precision.HIGHEST)
    o_ref[...] = jnp.maximum(acc, jnp.float32(0.0))


def _tc_matmul(self_rows, nsum_rows, weight):
    return pl.pallas_call(
        _tc_body,
        grid=(_NBT,),
        in_specs=[
            pl.BlockSpec((_BT, _D), lambda i: (i, 0)),
            pl.BlockSpec((_BT, _D), lambda i: (i, 0)),
            pl.BlockSpec((_E, 2 * _D), lambda i: (0, 0)),
        ],
        out_specs=pl.BlockSpec((_E, _BT), lambda i: (0, i)),
        out_shape=jax.ShapeDtypeStruct((_E, _B), jnp.float32),
    )(self_rows, nsum_rows, weight)


def kernel(features, nodes, neigh_idx, weight):
    nidx = neigh_idx.reshape(-1)
    self_rows, nsum_rows = _sc_gather_sum(features, nidx, nodes)
    return _tc_matmul(self_rows, nsum_rows, weight)
